# Initial kernel scaffold; baseline (speedup 1.0000x reference)
import jax
import jax.numpy as jnp
from jax.experimental import pallas as pl


def kernel(text, embed_table, U_i, U_f, U_c, U_o, V_i, V_f, V_c, V_o, b_i, b_f, b_c, b_o, W_dense, b_dense):
    raise NotImplementedError("write your pallas kernel here")


# speedup vs baseline: 2.3914x; 2.3914x over previous
import jax
import jax.numpy as jnp
from jax.experimental import pallas as pl
from jax.experimental.pallas import tpu as pltpu


def _proj_kernel(e_ref, u_ref, b_ref, o_ref):
    o_ref[...] = (
        jnp.dot(e_ref[...], u_ref[...], preferred_element_type=jnp.float32)
        + b_ref[...]
    )


def _scan_kernel(text_ref, p_hbm, v_hbm, wd_ref, bd_ref, o_ref,
                 v_vmem, gbuf, h_ref, c_ref, sem_g, sem_v):
    t = pl.program_id(0)
    S = pl.num_programs(0)
    B, H4 = gbuf.shape[1], gbuf.shape[2]
    H = H4 // 4

    @pl.when(t == 0)
    def _():
        pltpu.make_async_copy(v_hbm, v_vmem, sem_v).start()
        for i in range(B):
            tok = text_ref[i]
            pltpu.make_async_copy(p_hbm.at[tok], gbuf.at[0, i], sem_g.at[0]).start()
        pltpu.make_async_copy(v_hbm, v_vmem, sem_v).wait()
        h_ref[...] = jnp.zeros_like(h_ref)
        c_ref[...] = jnp.zeros_like(c_ref)

    @pl.when(t + 1 < S)
    def _():
        nxt = t + 1
        slot_n = nxt % 2
        for i in range(B):
            tok = text_ref[nxt * B + i]
            pltpu.make_async_copy(p_hbm.at[tok], gbuf.at[slot_n, i],
                                  sem_g.at[slot_n]).start()

    slot = t % 2
    for i in range(B):
        pltpu.make_async_copy(p_hbm.at[0], gbuf.at[slot, i], sem_g.at[slot]).wait()

    xu = gbuf[slot]
    gates = xu + jnp.dot(h_ref[...], v_vmem[...],
                         preferred_element_type=jnp.float32)
    i_g = jax.nn.sigmoid(gates[:, 0 * H:1 * H])
    f_g = jax.nn.sigmoid(gates[:, 1 * H:2 * H])
    g_g = jnp.tanh(gates[:, 2 * H:3 * H])
    o_g = jax.nn.sigmoid(gates[:, 3 * H:4 * H])
    c_new = f_g * c_ref[...] + i_g * g_g
    h_new = o_g * jnp.tanh(c_new)
    c_ref[...] = c_new
    h_ref[...] = h_new

    @pl.when(t == S - 1)
    def _():
        o_ref[...] = (
            jnp.dot(h_new, wd_ref[...], preferred_element_type=jnp.float32)
            + bd_ref[...]
        )


def kernel(text, embed_table, U_i, U_f, U_c, U_o, V_i, V_f, V_c, V_o,
           b_i, b_f, b_c, b_o, W_dense, b_dense):
    VOCAB, E = embed_table.shape
    H = V_i.shape[0]
    B, S = text.shape
    H4 = 4 * H
    POL = W_dense.shape[1]

    U_cat = jnp.concatenate([U_i, U_f, U_c, U_o], axis=1)
    b_cat = jnp.concatenate([b_i, b_f, b_c, b_o], axis=0).reshape(1, H4)
    V_cat = jnp.concatenate([V_i, V_f, V_c, V_o], axis=1)

    M_BLK = 256
    grid_a = VOCAB // M_BLK
    P = pl.pallas_call(
        _proj_kernel,
        out_shape=jax.ShapeDtypeStruct((VOCAB, H4), jnp.float32),
        grid=(grid_a,),
        in_specs=[
            pl.BlockSpec((M_BLK, E), lambda i: (i, 0)),
            pl.BlockSpec((E, H4), lambda i: (0, 0)),
            pl.BlockSpec((1, H4), lambda i: (0, 0)),
        ],
        out_specs=pl.BlockSpec((M_BLK, H4), lambda i: (i, 0)),
        compiler_params=pltpu.CompilerParams(
            dimension_semantics=("parallel",),
        ),
        name="vocab_gate_proj",
    )(embed_table, U_cat, b_cat)

    text_t = text.T.astype(jnp.int32).reshape(S * B)
    Wp = jnp.pad(W_dense.astype(jnp.float32), ((0, 0), (0, 128 - POL)))
    bp = jnp.pad(b_dense.astype(jnp.float32), (0, 128 - POL)).reshape(1, 128)

    out128 = pl.pallas_call(
        _scan_kernel,
        out_shape=jax.ShapeDtypeStruct((B, 128), jnp.float32),
        grid=(S,),
        in_specs=[
            pl.BlockSpec(memory_space=pltpu.SMEM),
            pl.BlockSpec(memory_space=pl.ANY),
            pl.BlockSpec(memory_space=pl.ANY),
            pl.BlockSpec((H, 128), lambda t: (0, 0)),
            pl.BlockSpec((1, 128), lambda t: (0, 0)),
        ],
        out_specs=pl.BlockSpec((B, 128), lambda t: (0, 0)),
        scratch_shapes=[
            pltpu.VMEM((H, H4), jnp.float32),
            pltpu.VMEM((2, B, H4), jnp.float32),
            pltpu.VMEM((B, H), jnp.float32),
            pltpu.VMEM((B, H), jnp.float32),
            pltpu.SemaphoreType.DMA((2,)),
            pltpu.SemaphoreType.DMA,
        ],
        compiler_params=pltpu.CompilerParams(
            dimension_semantics=("arbitrary",),
        ),
        name="lstm_scan",
    )(text_t, P, V_cat, Wp, bp)

    return out128[:, :POL]
